# baseline (speedup 1.0000x reference)
import functools

import jax
import jax.numpy as jnp
from jax import lax
from jax.experimental import pallas as pl
from jax.experimental.pallas import tpu as pltpu
from jax.experimental.pallas import tpu_sc as plsc

EDGES = 320000
DIM = 128
NTYPES = 256
NC = 2
NS = 16
NW = NC * NS
NSLAB = 5
SLAB_E = EDGES // NSLAB
ROWS_W = SLAB_E // NW
CHUNK = 80
NCHUNK = ROWS_W // CHUNK
NBUF = 5

BE = 6400
NB_SLAB = SLAB_E // BE


def _sc_gather_slab(node_feat, src, slab):
  mesh = plsc.VectorSubcoreMesh(core_axis_name="c", subcore_axis_name="s")

  @functools.partial(
      pl.kernel,
      mesh=mesh,
      out_type=jax.ShapeDtypeStruct((SLAB_E, DIM), jnp.float32),
      scratch_types=[
          pltpu.VMEM((ROWS_W,), jnp.int32),
      ] + [pltpu.VMEM((CHUNK, DIM), jnp.float32)] * NBUF
        + [pltpu.SemaphoreType.DMA] * (2 * NBUF),
  )
  def k(nf_hbm, src_hbm, s1_hbm, idx_all, *bufs_sems):
    bufs = bufs_sems[:NBUF]
    gsems = bufs_sems[NBUF:2 * NBUF]
    ssems = bufs_sems[2 * NBUF:]
    wid = lax.axis_index("s") * NC + lax.axis_index("c")
    lbase = wid * ROWS_W
    gbase = slab * SLAB_E + lbase

    pltpu.sync_copy(src_hbm.at[pl.ds(gbase, ROWS_W)], idx_all)

    def gfire(c, b):
      pltpu.async_copy(
          nf_hbm.at[idx_all.at[pl.ds(c * CHUNK, CHUNK)]], bufs[b], gsems[b])

    def gwait(c, b):
      pltpu.make_async_copy(
          nf_hbm.at[idx_all.at[pl.ds(c * CHUNK, CHUNK)]], bufs[b],
          gsems[b]).wait()

    def sfire(c, b):
      pltpu.async_copy(
          bufs[b], s1_hbm.at[pl.ds(lbase + c * CHUNK, CHUNK)], ssems[b])

    def swait(c, b):
      pltpu.make_async_copy(
          bufs[b], s1_hbm.at[pl.ds(lbase + c * CHUNK, CHUNK)],
          ssems[b]).wait()

    gfire(0, 0)
    gfire(1, 1)
    gfire(2, 2)
    gwait(0, 0); sfire(0, 0); gfire(3, 3)
    gwait(1, 1); sfire(1, 1); gfire(4, 4)

    def body(j, carry):
      c0 = 2 + NBUF * j
      for b2 in range(NBUF):
        c = c0 + b2
        b = (2 + b2) % NBUF
        gwait(c, b); sfire(c, b)
        swait(c - 2, b2); gfire(c + 3, b2)
      return carry

    lax.fori_loop(0, (NCHUNK - NBUF) // NBUF, body, 0)

    c = NCHUNK - 3
    gwait(c, c % NBUF); sfire(c, c % NBUF); swait(c - 2, (c - 2) % NBUF)
    c = NCHUNK - 2
    gwait(c, c % NBUF); sfire(c, c % NBUF); swait(c - 2, (c - 2) % NBUF)
    c = NCHUNK - 1
    gwait(c, c % NBUF); sfire(c, c % NBUF); swait(c - 2, (c - 2) % NBUF)
    swait(NCHUNK - 2, (NCHUNK - 2) % NBUF)
    swait(NCHUNK - 1, (NCHUNK - 1) % NBUF)

  return k(node_feat, src)


def _tc_slab(h_acc, et2, ef, s1, emb, W, b2, slab):
  def body(*refs):
    if h_acc is None:
      et_ref, ef_ref, s1_ref, emb_ref, w_ref, b_ref, o_ref = refs
    else:
      _, et_ref, ef_ref, s1_ref, emb_ref, w_ref, b_ref, o_ref = refs
    onehot = (et_ref[...] == lax.broadcasted_iota(
        jnp.int32, (1, NTYPES), 1)).astype(jnp.float32)
    rel = lax.dot_general(onehot, emb_ref[...], (((1,), (0,)), ((), ())),
                          preferred_element_type=jnp.float32)
    msg = ef_ref[...] + s1_ref[...] + rel
    acc = lax.dot_general(msg, w_ref[...], (((1,), (1,)), ((), ())),
                          preferred_element_type=jnp.float32)
    o_ref[...] = jnp.maximum(acc + b_ref[...], 0.0)

  off = slab * NB_SLAB
  data_specs = [
      pl.BlockSpec((BE, 1), lambda i: (off + i, 0)),
      pl.BlockSpec((BE, DIM), lambda i: (off + i, 0)),
      pl.BlockSpec((BE, DIM), lambda i: (i, 0)),
      pl.BlockSpec((NTYPES, DIM), lambda i: (0, 0)),
      pl.BlockSpec((DIM, DIM), lambda i: (0, 0)),
      pl.BlockSpec((1, DIM), lambda i: (0, 0)),
  ]
  if h_acc is None:
    in_specs, aliases, args = data_specs, {}, (et2, ef, s1, emb, W, b2)
  else:
    in_specs = [pl.BlockSpec(memory_space=pltpu.MemorySpace.HBM)] + data_specs
    aliases = {0: 0}
    args = (h_acc, et2, ef, s1, emb, W, b2)
  return pl.pallas_call(
      body,
      grid=(NB_SLAB,),
      in_specs=in_specs,
      out_specs=pl.BlockSpec((BE, DIM), lambda i: (off + i, 0)),
      out_shape=jax.ShapeDtypeStruct((EDGES, DIM), jnp.float32),
      input_output_aliases=aliases,
  )(*args)


def kernel(node_feat, src, edge_type, edge_feat, edge_emb, W, b):
  et2 = edge_type.reshape(EDGES, 1)
  b2 = b.reshape(1, DIM)
  s1 = [_sc_gather_slab(node_feat, src, s) for s in range(NSLAB)]
  h = None
  for s in range(NSLAB):
    h = _tc_slab(h, et2, edge_feat, s1[s], edge_emb, W, b2, s)
  return h


# speedup vs baseline: 1.0063x; 1.0063x over previous
import functools

import jax
import jax.numpy as jnp
from jax import lax
from jax.experimental import pallas as pl
from jax.experimental.pallas import tpu as pltpu
from jax.experimental.pallas import tpu_sc as plsc

EDGES = 320000
DIM = 128
NTYPES = 256
NC = 2
NS = 16
NW = NC * NS
NSLAB = 5
SLAB_E = EDGES // NSLAB
ROWS_W = SLAB_E // NW
CHUNK = 80
NCHUNK = ROWS_W // CHUNK
NBUF = 5

BE = 6400
NB_SLAB = SLAB_E // BE


def _sc_gather_slab(node_feat, src, slab):
  mesh = plsc.VectorSubcoreMesh(core_axis_name="c", subcore_axis_name="s")

  @functools.partial(
      pl.kernel,
      mesh=mesh,
      out_type=jax.ShapeDtypeStruct((SLAB_E, DIM), jnp.float32),
      scratch_types=[
          pltpu.VMEM((ROWS_W,), jnp.int32),
      ] + [pltpu.VMEM((CHUNK, DIM), jnp.float32)] * NBUF
        + [pltpu.SemaphoreType.DMA] * (2 * NBUF),
  )
  def k(nf_hbm, src_hbm, s1_hbm, idx_all, *bufs_sems):
    bufs = bufs_sems[:NBUF]
    gsems = bufs_sems[NBUF:2 * NBUF]
    ssems = bufs_sems[2 * NBUF:]
    wid = lax.axis_index("s") * NC + lax.axis_index("c")
    lbase = wid * ROWS_W
    gbase = slab * SLAB_E + lbase

    pltpu.sync_copy(src_hbm.at[pl.ds(gbase, ROWS_W)], idx_all)

    def gfire(c, b):
      pltpu.async_copy(
          nf_hbm.at[idx_all.at[pl.ds(c * CHUNK, CHUNK)]], bufs[b], gsems[b])

    def gwait(c, b):
      pltpu.make_async_copy(
          nf_hbm.at[idx_all.at[pl.ds(c * CHUNK, CHUNK)]], bufs[b],
          gsems[b]).wait()

    def sfire(c, b):
      pltpu.async_copy(
          bufs[b], s1_hbm.at[pl.ds(lbase + c * CHUNK, CHUNK)], ssems[b])

    def swait(c, b):
      pltpu.make_async_copy(
          bufs[b], s1_hbm.at[pl.ds(lbase + c * CHUNK, CHUNK)],
          ssems[b]).wait()

    gfire(0, 0)
    gfire(1, 1)
    gfire(2, 2)
    gwait(0, 0); sfire(0, 0); gfire(3, 3)
    gwait(1, 1); sfire(1, 1); gfire(4, 4)

    def body(j, carry):
      c0 = 2 + NBUF * j
      for b2 in range(NBUF):
        c = c0 + b2
        b = (2 + b2) % NBUF
        gwait(c, b); sfire(c, b)
        swait(c - 2, b2); gfire(c + 3, b2)
      return carry

    lax.fori_loop(0, (NCHUNK - NBUF) // NBUF, body, 0)

    c = NCHUNK - 3
    gwait(c, c % NBUF); sfire(c, c % NBUF); swait(c - 2, (c - 2) % NBUF)
    c = NCHUNK - 2
    gwait(c, c % NBUF); sfire(c, c % NBUF); swait(c - 2, (c - 2) % NBUF)
    c = NCHUNK - 1
    gwait(c, c % NBUF); sfire(c, c % NBUF); swait(c - 2, (c - 2) % NBUF)
    swait(NCHUNK - 2, (NCHUNK - 2) % NBUF)
    swait(NCHUNK - 1, (NCHUNK - 1) % NBUF)

  return k(node_feat, src)


def _tc_slab(h_acc, et2, ef, s1, emb, W, b2, slab):
  def body(*refs):
    if h_acc is None:
      et_ref, ef_ref, s1_ref, emb_ref, w_ref, b_ref, o_ref = refs
    else:
      _, et_ref, ef_ref, s1_ref, emb_ref, w_ref, b_ref, o_ref = refs
    onehot = (et_ref[...] == lax.broadcasted_iota(
        jnp.int32, (1, NTYPES), 1)).astype(jnp.bfloat16)
    rel = lax.dot_general(onehot, emb_ref[...], (((1,), (0,)), ((), ())),
                          preferred_element_type=jnp.float32)
    msg = (ef_ref[...] + s1_ref[...] + rel).astype(jnp.bfloat16)
    acc = lax.dot_general(msg, w_ref[...], (((1,), (1,)), ((), ())),
                          preferred_element_type=jnp.float32)
    o_ref[...] = jnp.maximum(acc + b_ref[...], 0.0)

  off = slab * NB_SLAB
  data_specs = [
      pl.BlockSpec((BE, 1), lambda i: (off + i, 0)),
      pl.BlockSpec((BE, DIM), lambda i: (off + i, 0)),
      pl.BlockSpec((BE, DIM), lambda i: (i, 0)),
      pl.BlockSpec((NTYPES, DIM), lambda i: (0, 0)),
      pl.BlockSpec((DIM, DIM), lambda i: (0, 0)),
      pl.BlockSpec((1, DIM), lambda i: (0, 0)),
  ]
  if h_acc is None:
    in_specs, aliases, args = data_specs, {}, (et2, ef, s1, emb, W, b2)
  else:
    in_specs = [pl.BlockSpec(memory_space=pltpu.MemorySpace.HBM)] + data_specs
    aliases = {0: 0}
    args = (h_acc, et2, ef, s1, emb, W, b2)
  return pl.pallas_call(
      body,
      grid=(NB_SLAB,),
      in_specs=in_specs,
      out_specs=pl.BlockSpec((BE, DIM), lambda i: (off + i, 0)),
      out_shape=jax.ShapeDtypeStruct((EDGES, DIM), jnp.float32),
      input_output_aliases=aliases,
  )(*args)


def kernel(node_feat, src, edge_type, edge_feat, edge_emb, W, b):
  et2 = edge_type.reshape(EDGES, 1)
  b2 = b.reshape(1, DIM)
  s1 = [_sc_gather_slab(node_feat, src, s) for s in range(NSLAB)]
  emb16 = edge_emb.astype(jnp.bfloat16)
  w16 = W.astype(jnp.bfloat16)
  h = None
  for s in range(NSLAB):
    h = _tc_slab(h, et2, edge_feat, s1[s], emb16, w16, b2, s)
  return h
